# speedup vs baseline: 8.6966x; 8.6966x over previous
import jax
import jax.numpy as jnp
from jax.experimental import pallas as pl

N = 1024
H = 64
B = 8


def _agg_gru_kernel(e_ref, adjf_ref, nft_ref, s_ref, h_ref,
                    wih_ref, whh_ref, bih_ref, bhh_ref, out_ref):
    p = e_ref[:] * adjf_ref[:] * nft_ref[:]
    agg = jnp.dot(s_ref[:], p, preferred_element_type=jnp.float32)
    h = h_ref[:]
    gi = jnp.dot(agg, wih_ref[:], preferred_element_type=jnp.float32) + bih_ref[:]
    gh = jnp.dot(h, whh_ref[:], preferred_element_type=jnp.float32) + bhh_ref[:]
    r = jax.nn.sigmoid(gi[:, :H] + gh[:, :H])
    z = jax.nn.sigmoid(gi[:, H:2 * H] + gh[:, H:2 * H])
    n = jnp.tanh(gi[:, 2 * H:] + r * gh[:, 2 * H:])
    out_ref[:] = (1.0 - z) * n + z * h


def kernel(node_features, adjacency_matrix, edge_embedding,
           weight_ih, weight_hh, bias_ih, bias_hh):
    adjf = (adjacency_matrix.reshape(N * N, 1) != 0).astype(jnp.float32)
    nft = jnp.tile(node_features, (B, 1))
    s = (jax.lax.broadcasted_iota(jnp.int32, (B, B * N), 1) // N
         == jax.lax.broadcasted_iota(jnp.int32, (B, B * N), 0)).astype(jnp.float32)
    wih_t = weight_ih.T
    whh_t = weight_hh.T
    bih = bias_ih.reshape(1, 3 * H)
    bhh = bias_hh.reshape(1, 3 * H)

    return pl.pallas_call(
        _agg_gru_kernel,
        grid=(N // B,),
        in_specs=[
            pl.BlockSpec((B * N, H), lambda i: (i, 0)),
            pl.BlockSpec((B * N, 1), lambda i: (i, 0)),
            pl.BlockSpec((B * N, H), lambda i: (0, 0)),
            pl.BlockSpec((B, B * N), lambda i: (0, 0)),
            pl.BlockSpec((B, H), lambda i: (i, 0)),
            pl.BlockSpec((H, 3 * H), lambda i: (0, 0)),
            pl.BlockSpec((H, 3 * H), lambda i: (0, 0)),
            pl.BlockSpec((1, 3 * H), lambda i: (0, 0)),
            pl.BlockSpec((1, 3 * H), lambda i: (0, 0)),
        ],
        out_specs=pl.BlockSpec((B, H), lambda i: (i, 0)),
        out_shape=jax.ShapeDtypeStruct((N, H), jnp.float32),
    )(edge_embedding, adjf, nft, s, node_features, wih_t, whh_t, bih, bhh)


# speedup vs baseline: 13.7342x; 1.5792x over previous
import jax
import jax.numpy as jnp
from jax.experimental import pallas as pl

N = 1024
H = 64
B = 8


def _agg_gru_kernel(e_ref, nft_ref, sa_ref, h_ref,
                    wih_ref, whh_ref, bih_ref, bhh_ref, out_ref):
    q = e_ref[:] * nft_ref[:]
    agg = jnp.dot(sa_ref[0], q, preferred_element_type=jnp.float32)
    h = h_ref[:]
    gi = jnp.dot(agg, wih_ref[:], preferred_element_type=jnp.float32) + bih_ref[:]
    gh = jnp.dot(h, whh_ref[:], preferred_element_type=jnp.float32) + bhh_ref[:]
    r = jax.nn.sigmoid(gi[:, :H] + gh[:, :H])
    z = jax.nn.sigmoid(gi[:, H:2 * H] + gh[:, H:2 * H])
    n = jnp.tanh(gi[:, 2 * H:] + r * gh[:, 2 * H:])
    out_ref[:] = (1.0 - z) * n + z * h


def kernel(node_features, adjacency_matrix, edge_embedding,
           weight_ih, weight_hh, bias_ih, bias_hh):
    a_f = (adjacency_matrix != 0).astype(jnp.float32)
    eye = jnp.eye(B, dtype=jnp.float32)
    sa = (a_f.reshape(N // B, B, 1, N) * eye[None, :, :, None]
          ).reshape(N // B, B, B * N)
    nft = jnp.tile(node_features, (B, 1))
    wih_t = weight_ih.T
    whh_t = weight_hh.T
    bih = bias_ih.reshape(1, 3 * H)
    bhh = bias_hh.reshape(1, 3 * H)

    return pl.pallas_call(
        _agg_gru_kernel,
        grid=(N // B,),
        in_specs=[
            pl.BlockSpec((B * N, H), lambda i: (i, 0)),
            pl.BlockSpec((B * N, H), lambda i: (0, 0)),
            pl.BlockSpec((1, B, B * N), lambda i: (i, 0, 0)),
            pl.BlockSpec((B, H), lambda i: (i, 0)),
            pl.BlockSpec((H, 3 * H), lambda i: (0, 0)),
            pl.BlockSpec((H, 3 * H), lambda i: (0, 0)),
            pl.BlockSpec((1, 3 * H), lambda i: (0, 0)),
            pl.BlockSpec((1, 3 * H), lambda i: (0, 0)),
        ],
        out_specs=pl.BlockSpec((B, H), lambda i: (i, 0)),
        out_shape=jax.ShapeDtypeStruct((N, H), jnp.float32),
    )(edge_embedding, nft, sa, node_features, wih_t, whh_t, bih, bhh)


# speedup vs baseline: 13.9903x; 1.0187x over previous
import jax
import jax.numpy as jnp
from jax.experimental import pallas as pl

N = 1024
H = 64
B = 8


def _agg_gru_kernel(e_ref, nft_ref, sa_ref, h_ref,
                    wih_ref, whh_ref, bih_ref, bhh_ref, out_ref):
    q = e_ref[:] * nft_ref[:]
    agg = jnp.dot(sa_ref[:], q, preferred_element_type=jnp.float32)
    h = h_ref[:]
    gi = jnp.dot(agg, wih_ref[:], preferred_element_type=jnp.float32) + bih_ref[:]
    gh = jnp.dot(h, whh_ref[:], preferred_element_type=jnp.float32) + bhh_ref[:]
    r = jax.nn.sigmoid(gi[:, :H] + gh[:, :H])
    z = jax.nn.sigmoid(gi[:, H:2 * H] + gh[:, H:2 * H])
    n = jnp.tanh(gi[:, 2 * H:] + r * gh[:, 2 * H:])
    out_ref[:] = (1.0 - z) * n + z * h


def kernel(node_features, adjacency_matrix, edge_embedding,
           weight_ih, weight_hh, bias_ih, bias_hh):
    a_f = (adjacency_matrix != 0).astype(jnp.float32)
    eye = jnp.eye(B, dtype=jnp.float32)
    sa = (a_f.reshape(N // B, B, 1, N) * eye[None, :, :, None]
          ).reshape(N, B * N)
    nft = jnp.tile(node_features, (B, 1))
    wih_t = weight_ih.T
    whh_t = weight_hh.T
    bih = bias_ih.reshape(1, 3 * H)
    bhh = bias_hh.reshape(1, 3 * H)

    return pl.pallas_call(
        _agg_gru_kernel,
        grid=(N // B,),
        in_specs=[
            pl.BlockSpec((B * N, H), lambda i: (i, 0)),
            pl.BlockSpec((B * N, H), lambda i: (0, 0)),
            pl.BlockSpec((B, B * N), lambda i: (i, 0)),
            pl.BlockSpec((B, H), lambda i: (i, 0)),
            pl.BlockSpec((H, 3 * H), lambda i: (0, 0)),
            pl.BlockSpec((H, 3 * H), lambda i: (0, 0)),
            pl.BlockSpec((1, 3 * H), lambda i: (0, 0)),
            pl.BlockSpec((1, 3 * H), lambda i: (0, 0)),
        ],
        out_specs=pl.BlockSpec((B, H), lambda i: (i, 0)),
        out_shape=jax.ShapeDtypeStruct((N, H), jnp.float32),
    )(edge_embedding, nft, sa, node_features, wih_t, whh_t, bih, bhh)
